# Initial kernel scaffold; baseline (speedup 1.0000x reference)
import jax
import jax.numpy as jnp
from jax.experimental import pallas as pl


def kernel(x, edge_index, W, u, c, bias, W1, b1, W2, b2):
    raise NotImplementedError("write your pallas kernel here")


# speedup vs baseline: 19.4204x; 19.4204x over previous
import functools

import jax
import jax.numpy as jnp
from jax import lax
from jax.experimental import pallas as pl
from jax.experimental.pallas import tpu as pltpu
from jax.experimental.pallas import tpu_sc as plsc

N = 10000
E = 320000
D = 128
H = 16
WID = 32
NC, NS = 2, 16
NT = NC * NS
EPT = E // NT
CH = 128
K = -(-EPT // CH)
EPAD = K * CH
NPAD = 10112
RPW = NPAD // NS


def _xwe_body(x_ref, wp_ref, e_ref, o_ref):
    o_ref[...] = (
        jnp.dot(x_ref[...], wp_ref[...], preferred_element_type=jnp.float32)
        + e_ref[...]
    )


_sc_mesh = plsc.VectorSubcoreMesh(core_axis_name="c", subcore_axis_name="s")


@functools.partial(
    pl.kernel,
    out_type=jax.ShapeDtypeStruct((NC, NPAD, WID), jnp.float32),
    mesh=_sc_mesh,
    scratch_types=[
        pltpu.VMEM((K, CH), jnp.int32),
        pltpu.VMEM((K, CH), jnp.int32),
        pltpu.VMEM((CH, WID), jnp.float32),
        pltpu.VMEM_SHARED((NPAD, WID), jnp.float32),
        pltpu.SemaphoreType.DMA,
    ],
    compiler_params=pltpu.CompilerParams(use_tc_tiling_on_sc=False),
)
def _edge_scatter(xwe_hbm, src_hbm, dst_hbm, zeros_hbm, out_hbm,
                  src_v, dst_v, rows_v, agg_sh, sem):
    c = lax.axis_index("c")
    s = lax.axis_index("s")
    t = c * NS + s
    pltpu.sync_copy(zeros_hbm.at[pl.ds(s * RPW, RPW)],
                    agg_sh.at[pl.ds(s * RPW, RPW)])
    plsc.subcore_barrier()
    pltpu.sync_copy(src_hbm.at[t], src_v)
    pltpu.sync_copy(dst_hbm.at[t], dst_v)

    def chunk(j, carry):
        pltpu.async_copy(xwe_hbm.at[src_v.at[j]], rows_v, sem).wait()
        pltpu.sync_copy(rows_v, agg_sh.at[dst_v.at[j]], add=True)
        return carry

    lax.fori_loop(0, K, chunk, 0)
    plsc.subcore_barrier()
    pltpu.sync_copy(agg_sh.at[pl.ds(s * RPW, RPW)],
                    out_hbm.at[c, pl.ds(s * RPW, RPW)])


def _mlp_body(p_ref, xwe_ref, bias_ref, w1_ref, b1_ref, w2_ref, b2_ref, o_ref):
    s = p_ref[0, :N, :] + p_ref[1, :N, :] + xwe_ref[...]
    agg = s[:, :H]
    cnt = s[:, H:H + 1]
    out = agg / jnp.maximum(cnt, 1.0) + bias_ref[...]
    h = jnp.maximum(out, 0.0)
    h = jnp.maximum(
        jnp.dot(h, w1_ref[...], preferred_element_type=jnp.float32)
        + b1_ref[...], 0.0)
    y = (jnp.dot(h, w2_ref[...], preferred_element_type=jnp.float32)
         + b2_ref[...])
    o_ref[...] = jax.nn.sigmoid(y)


def kernel(x, edge_index, W, u, c, bias, W1, b1, W2, b2):
    del u, c
    src = edge_index[0].astype(jnp.int32).reshape(NT, EPT)
    dst = edge_index[1].astype(jnp.int32).reshape(NT, EPT)
    pad_s = jnp.zeros((NT, EPAD - EPT), jnp.int32)
    pad_d = jnp.full((NT, EPAD - EPT), N, jnp.int32)
    srcp = jnp.concatenate([src, pad_s], axis=1).reshape(NT, K, CH)
    dstp = jnp.concatenate([dst, pad_d], axis=1).reshape(NT, K, CH)

    wp = jnp.pad(W, ((0, 0), (0, WID - H)))
    e_row = jnp.zeros((1, WID), jnp.float32).at[0, H].set(1.0)
    xwe = pl.pallas_call(
        _xwe_body,
        out_shape=jax.ShapeDtypeStruct((N, WID), jnp.float32),
    )(x, wp, e_row)

    zeros = jnp.zeros((NPAD, WID), jnp.float32)
    parts = _edge_scatter(xwe, srcp, dstp, zeros)

    y = pl.pallas_call(
        _mlp_body,
        out_shape=jax.ShapeDtypeStruct((N, 1), jnp.float32),
    )(parts, xwe, bias.reshape(1, H), W1, b1.reshape(1, 8),
      W2, b2.reshape(1, 1))
    return y
